# baseline (speedup 1.0000x reference)
import functools

import jax
import jax.numpy as jnp
from jax import lax
from jax.experimental import pallas as pl
from jax.experimental.pallas import tpu as pltpu
from jax.experimental.pallas import tpu_sc as plsc

N = 4096
E = 65536
Z_DIM = 64
ROW_BLK = 256

TOT = N * N
NW = 32
SLICE = TOT // NW
CHUNK = 16384
NCHUNK = SLICE // CHUNK
VPC = CHUNK // 16
CAP = E + 16
BATCH = 8
FLUSH = 512
BUFCAP = 768
DUMP = E + 8


def _gat_encode(x, edge_index, W, att_src, att_dst, bias):
    n = x.shape[0]
    h = x @ W.T
    loops = jnp.arange(n, dtype=edge_index.dtype)
    src = jnp.concatenate([edge_index[0], loops])
    dst = jnp.concatenate([edge_index[1], loops])
    a_src = (h * att_src).sum(-1)
    a_dst = (h * att_dst).sum(-1)
    alpha = a_src[src] + a_dst[dst]
    alpha = jax.nn.leaky_relu(alpha, 0.2)
    amax = jax.ops.segment_max(alpha, dst, num_segments=n)
    amax = jnp.where(jnp.isfinite(amax), amax, 0.0)
    ex = jnp.exp(alpha - amax[dst])
    denom = jax.ops.segment_sum(ex, dst, num_segments=n)
    coef = ex / (denom[dst] + 1e-16)
    out = jax.ops.segment_sum(coef[:, None] * h[src], dst, num_segments=n)
    return out + bias


def _matmul_kernel(zi_ref, z_ref, out_ref):
    s = jax.lax.dot_general(
        zi_ref[...], z_ref[...],
        dimension_numbers=(((1,), (1,)), ((), ())),
        preferred_element_type=jnp.float32,
    )
    out_ref[...] = jax.lax.bitcast_convert_type(jax.nn.sigmoid(s), jnp.int32)


def _dense_keys(z):
    return pl.pallas_call(
        _matmul_kernel,
        grid=(N // ROW_BLK,),
        in_specs=[
            pl.BlockSpec((ROW_BLK, Z_DIM), lambda i: (i, 0)),
            pl.BlockSpec((N, Z_DIM), lambda i: (0, 0)),
        ],
        out_specs=pl.BlockSpec((ROW_BLK, N), lambda i: (i, 0)),
        out_shape=jax.ShapeDtypeStruct((N, N), jnp.int32),
    )(z, z)


def _make_hist(nbins, shift, nfilt):
    mesh = plsc.VectorSubcoreMesh(core_axis_name="c", subcore_axis_name="s", num_cores=2)

    def body(keys_hbm, *rest):
        filt = rest[:nfilt]
        out_hbm = rest[nfilt]
        kbuf = rest[nfilt + 1]
        hist = rest[nfilt + 2]
        fbuf = rest[nfilt + 3]
        wid = lax.axis_index("s") * 2 + lax.axis_index("c")
        base = wid * SLICE

        fvals = []
        for fi in range(nfilt):
            pltpu.sync_copy(filt[fi], fbuf)
            fvals.append(fbuf[...])

        def zero_body(i, _):
            hist[pl.ds(i * 16, 16)] = jnp.zeros((16,), jnp.int32)
            return 0
        lax.fori_loop(0, nbins, zero_body, 0)

        lanes = lax.iota(jnp.int32, 16)

        def filt_mask(kv):
            m = None
            if nfilt >= 1:
                m = (lax.shift_right_logical(kv, 18) & 4095) == fvals[0]
            if nfilt >= 2:
                m = m & ((lax.shift_right_logical(kv, 6) & 4095) == fvals[1])
            return m

        def rmw(kv, inc):
            d = lax.shift_right_logical(kv, shift) & (nbins - 1)
            addr = d * 16 + lanes
            cur = plsc.load_gather(hist, [addr])
            plsc.store_scatter(hist, [addr], cur + inc)

        def chunk_body(ci, _):
            pltpu.sync_copy(keys_hbm.at[pl.ds(base + ci * CHUNK, CHUNK)], kbuf)

            if nfilt == 0:
                def vec_body(j, _):
                    kv = kbuf[pl.ds(j * 16, 16)]
                    rmw(kv, jnp.ones((16,), jnp.int32))
                    return 0
                lax.fori_loop(0, VPC, vec_body, 0)
            else:
                def batch_body(jj, _):
                    kvs = [kbuf[pl.ds((jj * BATCH + u) * 16, 16)]
                           for u in range(BATCH)]
                    ms = [filt_mask(kv) for kv in kvs]
                    accv = ms[0]
                    for u in range(1, BATCH):
                        accv = accv | ms[u]
                    anyp = plsc.all_reduce_population_count(accv)

                    def slow(_):
                        for u in range(BATCH):
                            rmw(kvs[u], ms[u].astype(jnp.int32))
                        return 0
                    return lax.cond(anyp[0] > 0, slow, lambda c: c, 0)
                lax.fori_loop(0, VPC // BATCH, batch_body, 0)
            return 0
        lax.fori_loop(0, NCHUNK, chunk_body, 0)

        pltpu.sync_copy(hist, out_hbm.at[wid])

    kern = functools.partial(
        pl.kernel, mesh=mesh,
        compiler_params=pltpu.CompilerParams(needs_layout_passes=False),
        out_type=jax.ShapeDtypeStruct((NW, nbins * 16), jnp.int32),
        scratch_types=[
            pltpu.VMEM((CHUNK,), jnp.int32),
            pltpu.VMEM((nbins * 16,), jnp.int32),
            pltpu.VMEM((16,), jnp.int32),
        ],
    )(body)
    return kern


_hist12_f0 = _make_hist(4096, 18, 0)
_hist12_f1 = _make_hist(4096, 6, 1)
_hist6_f2 = _make_hist(64, 0, 2)


def _emit_body(keys_hbm, tvec_hbm, gtb_hbm, eqb_hbm, neq_hbm,
               gtk_hbm, gti_hbm, eqi_hbm,
               kbuf, gk, gi, ei, xb, sv, sem):
    wid = lax.axis_index("s") * 2 + lax.axis_index("c")
    base = wid * SLICE
    lanes = lax.iota(jnp.int32, 16)

    pltpu.sync_copy(tvec_hbm, sv)
    tval = sv[...]
    pltpu.sync_copy(neq_hbm, sv)
    neq = sv[...]

    def pick(ref32):
        pltpu.sync_copy(ref32.at[pl.ds(0, 16)], sv)
        lo = jnp.sum(jnp.where(lanes == (wid & 15), sv[...], 0))
        pltpu.sync_copy(ref32.at[pl.ds(16, 16)], sv)
        hi = jnp.sum(jnp.where(lanes == (wid & 15), sv[...], 0))
        return jnp.where(wid < 16, lo, hi)

    gt_base = pick(gtb_hbm)
    eq_base = pick(eqb_hbm)

    def flush_gt(args):
        gcnt, gpos = args
        for t in range(FLUSH // 16):
            xb[pl.ds(t * 16, 16)] = gpos + (t * 16) + lanes
        c1 = pltpu.async_copy(gk.at[pl.ds(0, FLUSH)], gtk_hbm.at[xb], sem)
        c2 = pltpu.async_copy(gi.at[pl.ds(0, FLUSH)], gti_hbm.at[xb], sem)
        c1.wait()
        c2.wait()
        for t in range((BUFCAP - FLUSH) // 16):
            gk[pl.ds(t * 16, 16)] = gk[pl.ds(FLUSH + t * 16, 16)]
            gi[pl.ds(t * 16, 16)] = gi[pl.ds(FLUSH + t * 16, 16)]
        return gcnt - FLUSH, gpos + FLUSH

    def flush_eq(args):
        ecnt, epos = args
        for t in range(FLUSH // 16):
            pv = epos + (t * 16) + lanes
            xb[pl.ds(t * 16, 16)] = jnp.where(pv < neq, pv, DUMP)
        c1 = pltpu.async_copy(ei.at[pl.ds(0, FLUSH)], eqi_hbm.at[xb], sem)
        c1.wait()
        for t in range((BUFCAP - FLUSH) // 16):
            ei[pl.ds(t * 16, 16)] = ei[pl.ds(FLUSH + t * 16, 16)]
        return ecnt - FLUSH, epos + FLUSH

    def chunk_body(ci, carry):
        pltpu.sync_copy(keys_hbm.at[pl.ds(base + ci * CHUNK, CHUNK)], kbuf)
        cbase = base + ci * CHUNK

        def batch_body(jj, carry2):
            kvs = [kbuf[pl.ds((jj * BATCH + u) * 16, 16)] for u in range(BATCH)]
            mgts = [kv > tval for kv in kvs]
            meqs = [kv >= tval for kv in kvs]
            accv = meqs[0]
            for u in range(1, BATCH):
                accv = accv | meqs[u]
            anyp = plsc.all_reduce_population_count(accv)

            def slow(carry3):
                gcnt, gpos, ecnt, epos = carry3
                for u in range(BATCH):
                    fidx = cbase + (jj * BATCH + u) * 16 + lanes
                    meq = meqs[u] & (~mgts[u])
                    ng = plsc.all_reduce_population_count(mgts[u])[0]
                    plsc.store_compressed(gk.at[pl.ds(gcnt, 16)], kvs[u], mask=mgts[u])
                    plsc.store_compressed(gi.at[pl.ds(gcnt, 16)], fidx, mask=mgts[u])
                    gcnt = gcnt + ng
                    ne = plsc.all_reduce_population_count(meq)[0]
                    plsc.store_compressed(ei.at[pl.ds(ecnt, 16)], fidx, mask=meq)
                    ecnt = ecnt + ne
                gcnt, gpos = lax.cond(gcnt >= FLUSH, flush_gt, lambda a: a, (gcnt, gpos))
                ecnt, epos = lax.cond(ecnt >= FLUSH, flush_eq, lambda a: a, (ecnt, epos))
                return gcnt, gpos, ecnt, epos

            return lax.cond(anyp[0] > 0, slow, lambda c: c, carry2)

        return lax.fori_loop(0, VPC // BATCH, batch_body, carry)

    gcnt, gpos, ecnt, epos = lax.fori_loop(
        0, NCHUNK, chunk_body,
        (jnp.int32(0), gt_base, jnp.int32(0), eq_base))

    gcv = jnp.full((16,), 0, jnp.int32) + gcnt
    for t in range(FLUSH // 16):
        off = (t * 16) + lanes
        xb[pl.ds(t * 16, 16)] = jnp.where(off < gcv, gpos + off, DUMP)
    c1 = pltpu.async_copy(gk.at[pl.ds(0, FLUSH)], gtk_hbm.at[xb], sem)
    c2 = pltpu.async_copy(gi.at[pl.ds(0, FLUSH)], gti_hbm.at[xb], sem)
    c1.wait()
    c2.wait()
    ecv = jnp.full((16,), 0, jnp.int32) + ecnt
    for t in range(FLUSH // 16):
        off = (t * 16) + lanes
        pv = jnp.where(off < ecv, epos + off, DUMP)
        xb[pl.ds(t * 16, 16)] = jnp.where(pv < neq, pv, DUMP)
    c1 = pltpu.async_copy(ei.at[pl.ds(0, FLUSH)], eqi_hbm.at[xb], sem)
    c1.wait()


def _sc_emit(keys, tvec, gt_bases, eq_bases, neq_vec):
    mesh = plsc.VectorSubcoreMesh(core_axis_name="c", subcore_axis_name="s", num_cores=2)
    kern = functools.partial(
        pl.kernel, mesh=mesh,
        compiler_params=pltpu.CompilerParams(needs_layout_passes=False),
        out_type=(
            jax.ShapeDtypeStruct((CAP,), jnp.int32),
            jax.ShapeDtypeStruct((CAP,), jnp.int32),
            jax.ShapeDtypeStruct((CAP,), jnp.int32),
        ),
        scratch_types=[
            pltpu.VMEM((CHUNK,), jnp.int32),
            pltpu.VMEM((BUFCAP,), jnp.int32),
            pltpu.VMEM((BUFCAP,), jnp.int32),
            pltpu.VMEM((BUFCAP,), jnp.int32),
            pltpu.VMEM((FLUSH,), jnp.int32),
            pltpu.VMEM((16,), jnp.int32),
            pltpu.SemaphoreType.DMA,
        ],
    )(_emit_body)
    return kern(keys, tvec, gt_bases, eq_bases, neq_vec)


SR = 512
SC_ = 128


def _roll0(x, s):
    return jnp.concatenate([x[s:, :], x[:s, :]], axis=0)


def _roll1(x, s):
    return jnp.concatenate([x[:, s:], x[:, :s]], axis=1)


def _sort_kernel(cgt_ref, k_ref, i_ref, ok_ref, oi_ref):
    cgt = cgt_ref[0]
    rpos = lax.broadcasted_iota(jnp.int32, (SR, SC_), 0)
    cpos = lax.broadcasted_iota(jnp.int32, (SR, SC_), 1)
    fpos = cpos * SR + rpos
    valid = fpos < cgt
    key = jnp.where(valid, k_ref[...], 0)
    idx = jnp.where(valid, i_ref[...], 0)

    for p in range(1, 17):
        for sbit in range(p - 1, -1, -1):
            s = 1 << sbit
            if s < SR:
                pk = jnp.where((rpos & s) == 0, _roll0(key, s), _roll0(key, SR - s))
                pi = jnp.where((rpos & s) == 0, _roll0(idx, s), _roll0(idx, SR - s))
            else:
                m = s // SR
                pk = jnp.where((cpos & m) == 0, _roll1(key, m), _roll1(key, SC_ - m))
                pi = jnp.where((cpos & m) == 0, _roll1(idx, m), _roll1(idx, SC_ - m))
            upper = (fpos & s) != 0
            desc = (fpos & (1 << p)) == 0
            pbetter = (pk > key) | ((pk == key) & (pi < idx))
            take = (pbetter ^ upper) ^ desc
            take = ~take
            key = jnp.where(take, pk, key)
            idx = jnp.where(take, pi, idx)

    ok_ref[...] = key
    oi_ref[...] = idx


def _tc_sort(k2d, i2d, cgt_s):
    return pl.pallas_call(
        _sort_kernel,
        in_specs=[
            pl.BlockSpec(memory_space=pltpu.SMEM),
            pl.BlockSpec((SR, SC_), lambda: (0, 0)),
            pl.BlockSpec((SR, SC_), lambda: (0, 0)),
        ],
        out_specs=[
            pl.BlockSpec((SR, SC_), lambda: (0, 0)),
            pl.BlockSpec((SR, SC_), lambda: (0, 0)),
        ],
        out_shape=[
            jax.ShapeDtypeStruct((SR, SC_), jnp.int32),
            jax.ShapeDtypeStruct((SR, SC_), jnp.int32),
        ],
    )(cgt_s, k2d, i2d)


def _find_pivot(counts, rank):
    nb = counts.shape[0]
    rev = jnp.cumsum(counts[::-1])[::-1]
    gt = rev - counts
    sel = (gt < rank) & (rev >= rank)
    digit = jnp.argmax(sel).astype(jnp.int32)
    return digit, gt[digit]


def _bc16(x):
    return jnp.full((16,), x, jnp.int32)


def kernel(x, edge_index, node_index, W, att_src, att_dst, bias):
    z = _gat_encode(x, edge_index, W, att_src, att_dst, bias)
    keys = _dense_keys(z).reshape(-1)

    h1 = _hist12_f0(keys)
    h1t = h1.reshape(NW, 4096, 16).sum(2)
    c1 = h1t.sum(0)
    b1, above1 = _find_pivot(c1, jnp.int32(E))

    h2 = _hist12_f1(keys, _bc16(b1))
    h2t = h2.reshape(NW, 4096, 16).sum(2)
    c2 = h2t.sum(0)
    b2, above2 = _find_pivot(c2, E - above1)

    h3 = _hist6_f2(keys, _bc16(b1), _bc16(b2))
    h3t = h3.reshape(NW, 64, 16).sum(2)
    c3 = h3t.sum(0)
    b3, above3 = _find_pivot(c3, E - above1 - above2)

    tkey = (b1 << 18) | (b2 << 6) | b3
    c_gt = above1 + above2 + above3
    n_eq = E - c_gt

    d1 = jnp.arange(4096, dtype=jnp.int32)
    d3 = jnp.arange(64, dtype=jnp.int32)
    gt_w = (h1t @ (d1 > b1).astype(jnp.int32)
            + h2t @ (d1 > b2).astype(jnp.int32)
            + h3t @ (d3 > b3).astype(jnp.int32))
    eq_w = jnp.take_along_axis(h3t, jnp.full((NW, 1), b3), axis=1)[:, 0]
    gt_bases = jnp.cumsum(gt_w) - gt_w
    eq_bases = jnp.cumsum(eq_w) - eq_w

    gtk, gti, eqi = _sc_emit(
        keys, _bc16(tkey), gt_bases.astype(jnp.int32),
        eq_bases.astype(jnp.int32), _bc16(n_eq))

    k2d = gtk[:E].reshape(SC_, SR).T
    i2d = gti[:E].reshape(SC_, SR).T
    sk, si = _tc_sort(k2d, i2d, jnp.full((1,), c_gt, jnp.int32))
    sidx = si.T.reshape(-1)

    posn = jnp.arange(E, dtype=jnp.int32)
    eq_shift = jnp.roll(eqi[:E], c_gt)
    fin = jnp.where(posn < c_gt, sidx, eq_shift)
    src = lax.shift_right_logical(fin, 12).reshape(1, -1)
    dst = (fin & (N - 1)).reshape(1, -1)
    return jnp.concatenate([src, dst], axis=0)


# speedup vs baseline: 1.2716x; 1.2716x over previous
import functools

import jax
import jax.numpy as jnp
from jax import lax
from jax.experimental import pallas as pl
from jax.experimental.pallas import tpu as pltpu
from jax.experimental.pallas import tpu_sc as plsc

N = 4096
E = 65536
Z_DIM = 64
ROW_BLK = 256

TOT = N * N
NW = 32
SLICE = TOT // NW
CHUNK = 16384
NCHUNK = SLICE // CHUNK
VPC = CHUNK // 16
CAP = E + 16
BATCH = 8
FLUSH = 512
BUFCAP = 768
DUMP = E + 8


def _gat_encode(x, edge_index, W, att_src, att_dst, bias):
    n = x.shape[0]
    h = x @ W.T
    loops = jnp.arange(n, dtype=edge_index.dtype)
    src = jnp.concatenate([edge_index[0], loops])
    dst = jnp.concatenate([edge_index[1], loops])
    a_src = (h * att_src).sum(-1)
    a_dst = (h * att_dst).sum(-1)
    alpha = a_src[src] + a_dst[dst]
    alpha = jax.nn.leaky_relu(alpha, 0.2)
    amax = jax.ops.segment_max(alpha, dst, num_segments=n)
    amax = jnp.where(jnp.isfinite(amax), amax, 0.0)
    ex = jnp.exp(alpha - amax[dst])
    denom = jax.ops.segment_sum(ex, dst, num_segments=n)
    coef = ex / (denom[dst] + 1e-16)
    out = jax.ops.segment_sum(coef[:, None] * h[src], dst, num_segments=n)
    return out + bias


def _matmul_kernel(zi_ref, z_ref, out_ref):
    s = jax.lax.dot_general(
        zi_ref[...], z_ref[...],
        dimension_numbers=(((1,), (1,)), ((), ())),
        preferred_element_type=jnp.float32,
    )
    out_ref[...] = jax.lax.bitcast_convert_type(jax.nn.sigmoid(s), jnp.int32)


def _dense_keys(z):
    return pl.pallas_call(
        _matmul_kernel,
        grid=(N // ROW_BLK,),
        in_specs=[
            pl.BlockSpec((ROW_BLK, Z_DIM), lambda i: (i, 0)),
            pl.BlockSpec((N, Z_DIM), lambda i: (0, 0)),
        ],
        out_specs=pl.BlockSpec((ROW_BLK, N), lambda i: (i, 0)),
        out_shape=jax.ShapeDtypeStruct((N, N), jnp.int32),
    )(z, z)


def _make_hist(nbins, shift, nfilt):
    mesh = plsc.VectorSubcoreMesh(core_axis_name="c", subcore_axis_name="s", num_cores=2)

    def body(keys_hbm, *rest):
        filt = rest[:nfilt]
        out_hbm = rest[nfilt]
        kbuf = rest[nfilt + 1]
        hist = rest[nfilt + 2]
        fbuf = rest[nfilt + 3]
        wid = lax.axis_index("s") * 2 + lax.axis_index("c")
        base = wid * SLICE

        fvals = []
        for fi in range(nfilt):
            pltpu.sync_copy(filt[fi], fbuf)
            fvals.append(fbuf[...])

        def zero_body(i, _):
            hist[pl.ds(i * 16, 16)] = jnp.zeros((16,), jnp.int32)
            return 0
        lax.fori_loop(0, nbins, zero_body, 0)

        lanes = lax.iota(jnp.int32, 16)

        def filt_mask(kv):
            m = None
            if nfilt >= 1:
                m = (lax.shift_right_logical(kv, 18) & 4095) == fvals[0]
            if nfilt >= 2:
                m = m & ((lax.shift_right_logical(kv, 6) & 4095) == fvals[1])
            return m

        def rmw(kv, inc):
            d = lax.shift_right_logical(kv, shift) & (nbins - 1)
            addr = d * 16 + lanes
            cur = plsc.load_gather(hist, [addr])
            plsc.store_scatter(hist, [addr], cur + inc)

        def chunk_body(ci, _):
            pltpu.sync_copy(keys_hbm.at[pl.ds(base + ci * CHUNK, CHUNK)], kbuf)

            if nfilt == 0:
                def vec_body(j, _):
                    kv = kbuf[pl.ds(j * 16, 16)]
                    rmw(kv, jnp.ones((16,), jnp.int32))
                    return 0
                lax.fori_loop(0, VPC, vec_body, 0)
            else:
                def batch_body(jj, _):
                    kvs = [kbuf[pl.ds((jj * BATCH + u) * 16, 16)]
                           for u in range(BATCH)]
                    ms = [filt_mask(kv) for kv in kvs]
                    accv = ms[0]
                    for u in range(1, BATCH):
                        accv = accv | ms[u]
                    anyp = plsc.all_reduce_population_count(accv)

                    def slow(_):
                        for u in range(BATCH):
                            rmw(kvs[u], ms[u].astype(jnp.int32))
                        return 0
                    return lax.cond(anyp[0] > 0, slow, lambda c: c, 0)
                lax.fori_loop(0, VPC // BATCH, batch_body, 0)
            return 0
        lax.fori_loop(0, NCHUNK, chunk_body, 0)

        pltpu.sync_copy(hist, out_hbm.at[wid])

    kern = functools.partial(
        pl.kernel, mesh=mesh,
        compiler_params=pltpu.CompilerParams(needs_layout_passes=False),
        out_type=jax.ShapeDtypeStruct((NW, nbins * 16), jnp.int32),
        scratch_types=[
            pltpu.VMEM((CHUNK,), jnp.int32),
            pltpu.VMEM((nbins * 16,), jnp.int32),
            pltpu.VMEM((16,), jnp.int32),
        ],
    )(body)
    return kern


_hist12_f0 = _make_hist(4096, 18, 0)
_hist12_f1 = _make_hist(4096, 6, 1)
_hist6_f2 = _make_hist(64, 0, 2)


def _emit_body(keys_hbm, tvec_hbm, gtb_hbm, eqb_hbm, neq_hbm,
               gtk_hbm, gti_hbm, eqi_hbm,
               kbuf, gk, gi, ei, xb, sv, sem):
    wid = lax.axis_index("s") * 2 + lax.axis_index("c")
    base = wid * SLICE
    lanes = lax.iota(jnp.int32, 16)

    pltpu.sync_copy(tvec_hbm, sv)
    tval = sv[...]
    pltpu.sync_copy(neq_hbm, sv)
    neq = sv[...]

    def pick(ref32):
        pltpu.sync_copy(ref32.at[pl.ds(0, 16)], sv)
        lo = jnp.sum(jnp.where(lanes == (wid & 15), sv[...], 0))
        pltpu.sync_copy(ref32.at[pl.ds(16, 16)], sv)
        hi = jnp.sum(jnp.where(lanes == (wid & 15), sv[...], 0))
        return jnp.where(wid < 16, lo, hi)

    gt_base = pick(gtb_hbm)
    eq_base = pick(eqb_hbm)

    def chunk_body(ci, carry):
        gcnt, gpos, ecnt, epos = carry
        pltpu.sync_copy(keys_hbm.at[pl.ds(base + ci * CHUNK, CHUNK)], kbuf)
        cbase = base + ci * CHUNK

        def vec_body(j, carry2):
            gcnt, gpos, ecnt, epos = carry2
            kv = kbuf[pl.ds(j * 16, 16)]
            fidx = cbase + j * 16 + lanes
            mgt = kv > tval
            meq = kv == tval
            ng = plsc.all_reduce_population_count(mgt)[0]
            ne = plsc.all_reduce_population_count(meq)[0]

            plsc.store_compressed(gk.at[pl.ds(gcnt, 16)], kv, mask=mgt)
            plsc.store_compressed(gi.at[pl.ds(gcnt, 16)], fidx, mask=mgt)
            gcnt = gcnt + ng
            plsc.store_compressed(ei.at[pl.ds(ecnt, 16)], fidx, mask=meq)
            ecnt = ecnt + ne

            def flush_gt(args):
                gcnt, gpos = args
                pos = gpos + lanes
                pltpu.async_copy(gk.at[pl.ds(0, 16)], gtk_hbm.at[pos], sem).wait()
                pltpu.async_copy(gi.at[pl.ds(0, 16)], gti_hbm.at[pos], sem).wait()
                gk[pl.ds(0, 16)] = gk[pl.ds(16, 16)]
                gi[pl.ds(0, 16)] = gi[pl.ds(16, 16)]
                return gcnt - 16, gpos + 16

            gcnt, gpos = lax.cond(gcnt >= 16, flush_gt, lambda a: a, (gcnt, gpos))

            def flush_eq(args):
                ecnt, epos = args
                pos = epos + lanes
                pos = jnp.where(pos < neq, pos, DUMP)
                pltpu.async_copy(ei.at[pl.ds(0, 16)], eqi_hbm.at[pos], sem).wait()
                ei[pl.ds(0, 16)] = ei[pl.ds(16, 16)]
                return ecnt - 16, epos + 16

            ecnt, epos = lax.cond(ecnt >= 16, flush_eq, lambda a: a, (ecnt, epos))
            return gcnt, gpos, ecnt, epos

        return lax.fori_loop(0, VPC, vec_body, carry)

    gcnt, gpos, ecnt, epos = lax.fori_loop(
        0, NCHUNK, chunk_body,
        (jnp.int32(0), gt_base, jnp.int32(0), eq_base))

    pos = jnp.where(lanes < gcnt, gpos + lanes, DUMP)
    pltpu.async_copy(gk.at[pl.ds(0, 16)], gtk_hbm.at[pos], sem).wait()
    pltpu.async_copy(gi.at[pl.ds(0, 16)], gti_hbm.at[pos], sem).wait()
    pos = jnp.where(lanes < ecnt, epos + lanes, DUMP)
    pos = jnp.where(pos < neq, pos, DUMP)
    pltpu.async_copy(ei.at[pl.ds(0, 16)], eqi_hbm.at[pos], sem).wait()


def _sc_emit(keys, tvec, gt_bases, eq_bases, neq_vec):
    mesh = plsc.VectorSubcoreMesh(core_axis_name="c", subcore_axis_name="s", num_cores=2)
    kern = functools.partial(
        pl.kernel, mesh=mesh,
        compiler_params=pltpu.CompilerParams(needs_layout_passes=False),
        out_type=(
            jax.ShapeDtypeStruct((CAP,), jnp.int32),
            jax.ShapeDtypeStruct((CAP,), jnp.int32),
            jax.ShapeDtypeStruct((CAP,), jnp.int32),
        ),
        scratch_types=[
            pltpu.VMEM((CHUNK,), jnp.int32),
            pltpu.VMEM((BUFCAP,), jnp.int32),
            pltpu.VMEM((BUFCAP,), jnp.int32),
            pltpu.VMEM((BUFCAP,), jnp.int32),
            pltpu.VMEM((FLUSH,), jnp.int32),
            pltpu.VMEM((16,), jnp.int32),
            pltpu.SemaphoreType.DMA,
        ],
    )(_emit_body)
    return kern(keys, tvec, gt_bases, eq_bases, neq_vec)


SR = 512
SC_ = 128


def _roll0(x, s):
    return jnp.concatenate([x[s:, :], x[:s, :]], axis=0)


def _roll1(x, s):
    return jnp.concatenate([x[:, s:], x[:, :s]], axis=1)


def _sort_kernel(cgt_ref, k_ref, i_ref, ok_ref, oi_ref):
    cgt = cgt_ref[0]
    rpos = lax.broadcasted_iota(jnp.int32, (SR, SC_), 0)
    cpos = lax.broadcasted_iota(jnp.int32, (SR, SC_), 1)
    fpos = cpos * SR + rpos
    valid = fpos < cgt
    key = jnp.where(valid, k_ref[...], 0)
    idx = jnp.where(valid, i_ref[...], 0)

    for p in range(1, 17):
        for sbit in range(p - 1, -1, -1):
            s = 1 << sbit
            if s < SR:
                pk = jnp.where((rpos & s) == 0, _roll0(key, s), _roll0(key, SR - s))
                pi = jnp.where((rpos & s) == 0, _roll0(idx, s), _roll0(idx, SR - s))
            else:
                m = s // SR
                pk = jnp.where((cpos & m) == 0, _roll1(key, m), _roll1(key, SC_ - m))
                pi = jnp.where((cpos & m) == 0, _roll1(idx, m), _roll1(idx, SC_ - m))
            upper = (fpos & s) != 0
            desc = (fpos & (1 << p)) == 0
            pbetter = (pk > key) | ((pk == key) & (pi < idx))
            take = (pbetter ^ upper) ^ desc
            take = ~take
            key = jnp.where(take, pk, key)
            idx = jnp.where(take, pi, idx)

    ok_ref[...] = key
    oi_ref[...] = idx


def _tc_sort(k2d, i2d, cgt_s):
    return pl.pallas_call(
        _sort_kernel,
        in_specs=[
            pl.BlockSpec(memory_space=pltpu.SMEM),
            pl.BlockSpec((SR, SC_), lambda: (0, 0)),
            pl.BlockSpec((SR, SC_), lambda: (0, 0)),
        ],
        out_specs=[
            pl.BlockSpec((SR, SC_), lambda: (0, 0)),
            pl.BlockSpec((SR, SC_), lambda: (0, 0)),
        ],
        out_shape=[
            jax.ShapeDtypeStruct((SR, SC_), jnp.int32),
            jax.ShapeDtypeStruct((SR, SC_), jnp.int32),
        ],
    )(cgt_s, k2d, i2d)


def _find_pivot(counts, rank):
    nb = counts.shape[0]
    rev = jnp.cumsum(counts[::-1])[::-1]
    gt = rev - counts
    sel = (gt < rank) & (rev >= rank)
    digit = jnp.argmax(sel).astype(jnp.int32)
    return digit, gt[digit]


def _bc16(x):
    return jnp.full((16,), x, jnp.int32)


def kernel(x, edge_index, node_index, W, att_src, att_dst, bias):
    z = _gat_encode(x, edge_index, W, att_src, att_dst, bias)
    keys = _dense_keys(z).reshape(-1)

    h1 = _hist12_f0(keys)
    h1t = h1.reshape(NW, 4096, 16).sum(2)
    c1 = h1t.sum(0)
    b1, above1 = _find_pivot(c1, jnp.int32(E))

    h2 = _hist12_f1(keys, _bc16(b1))
    h2t = h2.reshape(NW, 4096, 16).sum(2)
    c2 = h2t.sum(0)
    b2, above2 = _find_pivot(c2, E - above1)

    h3 = _hist6_f2(keys, _bc16(b1), _bc16(b2))
    h3t = h3.reshape(NW, 64, 16).sum(2)
    c3 = h3t.sum(0)
    b3, above3 = _find_pivot(c3, E - above1 - above2)

    tkey = (b1 << 18) | (b2 << 6) | b3
    c_gt = above1 + above2 + above3
    n_eq = E - c_gt

    d1 = jnp.arange(4096, dtype=jnp.int32)
    d3 = jnp.arange(64, dtype=jnp.int32)
    gt_w = (h1t @ (d1 > b1).astype(jnp.int32)
            + h2t @ (d1 > b2).astype(jnp.int32)
            + h3t @ (d3 > b3).astype(jnp.int32))
    eq_w = jnp.take_along_axis(h3t, jnp.full((NW, 1), b3), axis=1)[:, 0]
    gt_bases = jnp.cumsum(gt_w) - gt_w
    eq_bases = jnp.cumsum(eq_w) - eq_w

    gtk, gti, eqi = _sc_emit(
        keys, _bc16(tkey), gt_bases.astype(jnp.int32),
        eq_bases.astype(jnp.int32), _bc16(n_eq))

    k2d = gtk[:E].reshape(SC_, SR).T
    i2d = gti[:E].reshape(SC_, SR).T
    sk, si = _tc_sort(k2d, i2d, jnp.full((1,), c_gt, jnp.int32))
    sidx = si.T.reshape(-1)

    posn = jnp.arange(E, dtype=jnp.int32)
    eq_shift = jnp.roll(eqi[:E], c_gt)
    fin = jnp.where(posn < c_gt, sidx, eq_shift)
    src = lax.shift_right_logical(fin, 12).reshape(1, -1)
    dst = (fin & (N - 1)).reshape(1, -1)
    return jnp.concatenate([src, dst], axis=0)
